# baseline (speedup 1.0000x reference)
import functools

import jax
import jax.numpy as jnp
from jax import lax
from jax.experimental import pallas as pl
from jax.experimental.pallas import tpu as pltpu
from jax.experimental.pallas import tpu_sc as plsc

B = 1024
D = 8
V = 100000

_NC = 2
_NS = 16
_NW = _NC * _NS
_B_PER_W = B // _NW

_TB = 32


@functools.lru_cache(maxsize=1)
def _make_sc_gather():
    mesh = plsc.VectorSubcoreMesh(core_axis_name="c", subcore_axis_name="s")

    @functools.partial(
        pl.kernel,
        mesh=mesh,
        out_type=jax.ShapeDtypeStruct((B, D), jnp.float32),
        scratch_types=[
            pltpu.VMEM((_B_PER_W,), jnp.int32),
            pltpu.VMEM((_B_PER_W, D), jnp.float32),
            pltpu.SemaphoreType.DMA,
        ],
        compiler_params=pltpu.CompilerParams(use_tc_tiling_on_sc=False),
    )
    def sc_gather(table_hbm, idx_hbm, out_hbm, idx_v, rows_v, sem):
        wid = lax.axis_index("s") * _NC + lax.axis_index("c")
        base = wid * _B_PER_W
        pltpu.sync_copy(idx_hbm.at[pl.ds(base, _B_PER_W)], idx_v)
        pltpu.async_copy(table_hbm.at[idx_v], rows_v, sem).wait()
        pltpu.sync_copy(rows_v, out_hbm.at[pl.ds(base, _B_PER_W)])

    return sc_gather


def _mm_body(emb_ref, w_ref, b_ref, out_ref):
    out_ref[...] = (
        jnp.dot(emb_ref[...], w_ref[...], preferred_element_type=jnp.float32)
        + b_ref[...]
    )


def _tc_project(emb, W, b2d):
    grid = (B // _TB,)
    return pl.pallas_call(
        _mm_body,
        grid=grid,
        in_specs=[
            pl.BlockSpec((_TB, D), lambda i: (i, 0)),
            pl.BlockSpec((D, V), lambda i: (0, 0)),
            pl.BlockSpec((1, V), lambda i: (0, 0)),
        ],
        out_specs=pl.BlockSpec((_TB, V), lambda i: (i, 0)),
        out_shape=jax.ShapeDtypeStruct((B, V), jnp.float32),
    )(emb, W, b2d)


def kernel(x, E, W, b):
    idx = x.astype(jnp.int32)
    emb = _make_sc_gather()(E, idx)
    return _tc_project(emb, W, b.reshape(1, V))


# speedup vs baseline: 1.0008x; 1.0008x over previous
import functools

import jax
import jax.numpy as jnp
from jax import lax
from jax.experimental import pallas as pl
from jax.experimental.pallas import tpu as pltpu
from jax.experimental.pallas import tpu_sc as plsc

B = 1024
D = 8
V = 100000

_NC = 2
_NS = 16
_NW = _NC * _NS
_B_PER_W = B // _NW

_TB = 16
_NBUF = 4
_NSTEP = B // _TB


@functools.lru_cache(maxsize=1)
def _make_sc_gather():
    mesh = plsc.VectorSubcoreMesh(core_axis_name="c", subcore_axis_name="s")

    @functools.partial(
        pl.kernel,
        mesh=mesh,
        out_type=jax.ShapeDtypeStruct((B, D), jnp.float32),
        scratch_types=[
            pltpu.VMEM((_B_PER_W,), jnp.int32),
            pltpu.VMEM((_B_PER_W, D), jnp.float32),
            pltpu.SemaphoreType.DMA,
        ],
        compiler_params=pltpu.CompilerParams(use_tc_tiling_on_sc=False),
    )
    def sc_gather(table_hbm, idx_hbm, out_hbm, idx_v, rows_v, sem):
        wid = lax.axis_index("s") * _NC + lax.axis_index("c")
        base = wid * _B_PER_W
        pltpu.sync_copy(idx_hbm.at[pl.ds(base, _B_PER_W)], idx_v)
        pltpu.async_copy(table_hbm.at[idx_v], rows_v, sem).wait()
        pltpu.sync_copy(rows_v, out_hbm.at[pl.ds(base, _B_PER_W)])

    return sc_gather


def _mm_body(emb_ref, w_ref, b_ref, out_hbm, buf, sems):
    i = pl.program_id(0)
    j = lax.rem(i, _NBUF)

    def _copy(step, k):
        return pltpu.make_async_copy(
            buf.at[k], out_hbm.at[pl.ds(step * _TB, _TB)], sems.at[k]
        )

    @pl.when(i >= _NBUF)
    def _():
        _copy(i - _NBUF, j).wait()

    buf[j] = (
        jnp.dot(emb_ref[...], w_ref[...], preferred_element_type=jnp.float32)
        + b_ref[...]
    )
    _copy(i, j).start()

    @pl.when(i == _NSTEP - 1)
    def _():
        for k in range(_NBUF):
            pltpu.make_async_copy(
                buf.at[k],
                out_hbm.at[pl.ds((_NSTEP - _NBUF + k) * _TB, _TB)],
                sems.at[k],
            ).wait()


def _tc_project(emb, W, b2d):
    grid = (_NSTEP,)
    return pl.pallas_call(
        _mm_body,
        grid=grid,
        in_specs=[
            pl.BlockSpec((_TB, D), lambda i: (i, 0)),
            pl.BlockSpec((D, V), lambda i: (0, 0)),
            pl.BlockSpec((1, V), lambda i: (0, 0)),
        ],
        out_specs=pl.BlockSpec(memory_space=pl.ANY),
        out_shape=jax.ShapeDtypeStruct((B, V), jnp.float32),
        scratch_shapes=[
            pltpu.VMEM((_NBUF, _TB, V), jnp.float32),
            pltpu.SemaphoreType.DMA((_NBUF,)),
        ],
    )(emb, W, b2d)


def kernel(x, E, W, b):
    idx = x.astype(jnp.int32)
    emb = _make_sc_gather()(E, idx)
    return _tc_project(emb, W, b.reshape(1, V))


# speedup vs baseline: 1.0632x; 1.0623x over previous
import functools

import jax
import jax.numpy as jnp
from jax import lax
from jax.experimental import pallas as pl
from jax.experimental.pallas import tpu as pltpu
from jax.experimental.pallas import tpu_sc as plsc

B = 1024
D = 8
V = 100000

_NC = 2
_NS = 16
_NW = _NC * _NS
_B_PER_W = B // _NW

_TB = 16
_NBUF = 4
_NSTEP = B // _TB


@functools.lru_cache(maxsize=1)
def _make_sc_gather():
    mesh = plsc.VectorSubcoreMesh(core_axis_name="c", subcore_axis_name="s")

    @functools.partial(
        pl.kernel,
        mesh=mesh,
        out_type=jax.ShapeDtypeStruct((B, D), jnp.float32),
        scratch_types=[
            pltpu.VMEM((_B_PER_W,), jnp.int32),
            pltpu.VMEM((_B_PER_W, D), jnp.float32),
            pltpu.SemaphoreType.DMA,
        ],
        compiler_params=pltpu.CompilerParams(use_tc_tiling_on_sc=False),
    )
    def sc_gather(table_hbm, idx_hbm, out_hbm, idx_v, rows_v, sem):
        wid = lax.axis_index("s") * _NC + lax.axis_index("c")
        base = wid * _B_PER_W
        pltpu.sync_copy(idx_hbm.at[pl.ds(base, _B_PER_W)], idx_v)
        pltpu.async_copy(table_hbm.at[idx_v], rows_v, sem).wait()
        pltpu.sync_copy(rows_v, out_hbm.at[pl.ds(base, _B_PER_W)])

    return sc_gather


def _mm_body(emb_ref, w_ref, b_ref, out_hbm, buf, sems):
    i = pl.program_id(0)
    j = lax.rem(i, _NBUF)

    def _copy(step, k):
        return pltpu.make_async_copy(
            buf.at[k], out_hbm.at[pl.ds(step * _TB, _TB)], sems.at[k]
        )

    @pl.when(i >= _NBUF)
    def _():
        _copy(i - _NBUF, j).wait()

    buf[j] = (
        jnp.dot(emb_ref[...], w_ref[...], preferred_element_type=jnp.float32)
        + b_ref[...]
    )
    _copy(i, j).start()

    @pl.when(i == _NSTEP - 1)
    def _():
        for k in range(_NBUF):
            pltpu.make_async_copy(
                buf.at[k],
                out_hbm.at[pl.ds((_NSTEP - _NBUF + k) * _TB, _TB)],
                sems.at[k],
            ).wait()


def _tc_project(emb, W, b2d):
    grid = (_NSTEP,)
    return pl.pallas_call(
        _mm_body,
        grid=grid,
        in_specs=[
            pl.BlockSpec((_TB, D), lambda i: (i, 0)),
            pl.BlockSpec((D, V), lambda i: (0, 0)),
            pl.BlockSpec((1, V), lambda i: (0, 0)),
        ],
        out_specs=pl.BlockSpec(memory_space=pl.ANY),
        out_shape=jax.ShapeDtypeStruct((B, V), jnp.float32),
        scratch_shapes=[
            pltpu.VMEM((_NBUF, _TB, V), jnp.float32),
            pltpu.SemaphoreType.DMA((_NBUF,)),
        ],
    )(emb, W, b2d)


def kernel(x, E, W, b):
    idx = x.astype(jnp.int32)
    emb = jnp.take(E, idx, axis=0)
    return _tc_project(emb, W, b.reshape(1, V))


# speedup vs baseline: 3.6470x; 3.4302x over previous
import functools

import jax
import jax.numpy as jnp
from jax import lax
from jax.experimental import pallas as pl
from jax.experimental.pallas import tpu as pltpu
from jax.experimental.pallas import tpu_sc as plsc

B = 1024
D = 8
V = 100000

_NC = 2
_NS = 16
_NW = _NC * _NS
_B_PER_W = B // _NW

_TV = 2048


@functools.lru_cache(maxsize=1)
def _make_sc_gather():
    mesh = plsc.VectorSubcoreMesh(core_axis_name="c", subcore_axis_name="s")

    n_elems = _B_PER_W * D

    @functools.partial(
        pl.kernel,
        mesh=mesh,
        out_type=jax.ShapeDtypeStruct((B * D,), jnp.float32),
        scratch_types=[
            pltpu.VMEM((_B_PER_W,), jnp.int32),
            pltpu.VMEM((n_elems,), jnp.int32),
            pltpu.VMEM((n_elems,), jnp.float32),
            pltpu.SemaphoreType.DMA,
        ],
        compiler_params=pltpu.CompilerParams(
            use_tc_tiling_on_sc=False, needs_layout_passes=False
        ),
    )
    def sc_gather(tflat_hbm, idx_hbm, out_hbm, idx_v, idx8_v, vals_v, sem):
        wid = lax.axis_index("s") * _NC + lax.axis_index("c")
        base = wid * _B_PER_W
        pltpu.sync_copy(idx_hbm.at[pl.ds(base, _B_PER_W)], idx_v)
        lanes = lax.iota(jnp.int32, 16)
        for c in range(n_elems // 16):
            p = lanes + (16 * c)
            row = lax.shift_right_logical(p, 3)
            dim = lax.bitwise_and(p, 7)
            xi = plsc.load_gather(idx_v, [row])
            idx8_v[pl.ds(16 * c, 16)] = xi + dim * V
        pltpu.async_copy(tflat_hbm.at[idx8_v], vals_v, sem).wait()
        pltpu.sync_copy(vals_v, out_hbm.at[pl.ds(base * D, n_elems)])

    return sc_gather


def _mm_body(w9_ref, e9_ref, out_ref):
    out_ref[...] = lax.dot_general(
        w9_ref[...],
        e9_ref[...],
        dimension_numbers=(((0,), (0,)), ((), ())),
        preferred_element_type=jnp.float32,
    )


def _tc_project(emb9T, W9):
    return pl.pallas_call(
        _mm_body,
        grid=(pl.cdiv(V, _TV),),
        in_specs=[
            pl.BlockSpec((D + 1, _TV), lambda i: (0, i)),
            pl.BlockSpec((D + 1, B), lambda i: (0, 0)),
        ],
        out_specs=pl.BlockSpec((_TV, B), lambda i: (i, 0)),
        out_shape=jax.ShapeDtypeStruct((V, B), jnp.float32),
    )(W9, emb9T)


def kernel(x, E, W, b):
    idx = x.astype(jnp.int32)
    emb = _make_sc_gather()(E.T.reshape(-1), idx).reshape(B, D)
    emb9T = jnp.concatenate(
        [emb.T, jnp.ones((1, B), jnp.float32)], axis=0
    )
    W9 = jnp.concatenate([W, b[None, :]], axis=0)
    return _tc_project(emb9T, W9).T


# speedup vs baseline: 3.6987x; 1.0142x over previous
import functools

import jax
import jax.numpy as jnp
from jax import lax
from jax.experimental import pallas as pl
from jax.experimental.pallas import tpu as pltpu
from jax.experimental.pallas import tpu_sc as plsc

B = 1024
D = 8
V = 100000

_NC = 2
_NS = 16
_NW = _NC * _NS
_B_PER_W = B // _NW

_TV = 2048


@functools.lru_cache(maxsize=1)
def _make_sc_gather():
    mesh = plsc.VectorSubcoreMesh(core_axis_name="c", subcore_axis_name="s")

    n_elems = _B_PER_W * D

    @functools.partial(
        pl.kernel,
        mesh=mesh,
        out_type=jax.ShapeDtypeStruct((B * D,), jnp.float32),
        scratch_types=[
            pltpu.VMEM((_B_PER_W,), jnp.int32),
            pltpu.VMEM((n_elems,), jnp.int32),
            pltpu.VMEM((n_elems,), jnp.float32),
            pltpu.SemaphoreType.DMA,
        ],
        compiler_params=pltpu.CompilerParams(
            use_tc_tiling_on_sc=False, needs_layout_passes=False
        ),
    )
    def sc_gather(tflat_hbm, idx_hbm, out_hbm, idx_v, idx8_v, vals_v, sem):
        wid = lax.axis_index("s") * _NC + lax.axis_index("c")
        base = wid * _B_PER_W
        pltpu.sync_copy(idx_hbm.at[pl.ds(base, _B_PER_W)], idx_v)
        lanes = lax.iota(jnp.int32, 16)
        for c in range(n_elems // 16):
            p = lanes + (16 * c)
            row = lax.shift_right_logical(p, 3)
            dim = lax.bitwise_and(p, 7)
            xi = plsc.load_gather(idx_v, [row])
            idx8_v[pl.ds(16 * c, 16)] = xi + dim * V
        pltpu.async_copy(tflat_hbm.at[idx8_v], vals_v, sem).wait()
        pltpu.sync_copy(vals_v, out_hbm.at[pl.ds(base * D, n_elems)])

    return sc_gather


def _mm_body(w_ref, b_ref, e9_ref, out_ref):
    w9 = jnp.concatenate([w_ref[...], b_ref[...]], axis=0)
    out_ref[...] = lax.dot_general(
        w9,
        e9_ref[...],
        dimension_numbers=(((0,), (0,)), ((), ())),
        preferred_element_type=jnp.float32,
    )


def _tc_project(emb9T, W, b2d):
    return pl.pallas_call(
        _mm_body,
        grid=(pl.cdiv(V, _TV),),
        in_specs=[
            pl.BlockSpec((D, _TV), lambda i: (0, i)),
            pl.BlockSpec((1, _TV), lambda i: (0, i)),
            pl.BlockSpec((D + 1, B), lambda i: (0, 0)),
        ],
        out_specs=pl.BlockSpec((_TV, B), lambda i: (i, 0)),
        out_shape=jax.ShapeDtypeStruct((V, B), jnp.float32),
    )(W, b2d, emb9T)


def kernel(x, E, W, b):
    idx = x.astype(jnp.int32)
    emb = _make_sc_gather()(E.T.reshape(-1), idx).reshape(B, D)
    emb9T = jnp.concatenate(
        [emb.T, jnp.ones((1, B), jnp.float32)], axis=0
    )
    return _tc_project(emb9T, W, b.reshape(1, V)).T
